# Initial kernel scaffold; baseline (speedup 1.0000x reference)
import jax
import jax.numpy as jnp
from jax.experimental import pallas as pl


def kernel(x, weight):
    raise NotImplementedError("write your pallas kernel here")


# speedup vs baseline: 1.1020x; 1.1020x over previous
import functools

import jax
import jax.numpy as jnp
from jax import lax
from jax.experimental import pallas as pl
from jax.experimental.pallas import tpu as pltpu
from jax.experimental.pallas import tpu_sc as plsc

NUM_EMB = 1000000
DIM = 32
TOTAL = 16384 * 50

_NC = 2
_NS = 16
_NW = _NC * _NS
_BPW = TOTAL // _NW
_CHUNK = 1024
_NCHUNK = _BPW // _CHUNK


def _emb_kernel(idx_hbm, table_hbm, out_hbm, idx_v, rows_v, sem_idx, sem_g):
    wid = lax.axis_index("s") * _NC + lax.axis_index("c")
    base = wid * _BPW
    pltpu.async_copy(idx_hbm.at[pl.ds(base, _BPW)], idx_v, sem_idx).wait()

    def body(j, carry):
        off = j * _CHUNK
        pltpu.async_copy(
            table_hbm.at[idx_v.at[pl.ds(off, _CHUNK)]], rows_v, sem_g
        ).wait()
        pltpu.async_copy(
            rows_v, out_hbm.at[pl.ds(base + off, _CHUNK)], sem_idx
        ).wait()
        return carry

    lax.fori_loop(0, _NCHUNK, body, 0)


@jax.jit
def _embedding_lookup(idx_flat, weight):
    mesh = plsc.VectorSubcoreMesh(core_axis_name="c", subcore_axis_name="s")
    f = functools.partial(
        pl.kernel,
        mesh=mesh,
        out_type=jax.ShapeDtypeStruct((TOTAL, DIM), jnp.float32),
        scratch_types=[
            pltpu.VMEM((_BPW,), jnp.int32),
            pltpu.VMEM((_CHUNK, DIM), jnp.float32),
            pltpu.SemaphoreType.DMA,
            pltpu.SemaphoreType.DMA,
        ],
        compiler_params=pltpu.CompilerParams(use_tc_tiling_on_sc=False),
    )(_emb_kernel)
    return f(idx_flat, weight)


def kernel(x, weight):
    idx_flat = x.reshape(-1).astype(jnp.int32)
    out = _embedding_lookup(idx_flat, weight)
    return out.reshape(x.shape[0], x.shape[1], DIM)


# speedup vs baseline: 1.1101x; 1.0073x over previous
import functools

import jax
import jax.numpy as jnp
from jax import lax
from jax.experimental import pallas as pl
from jax.experimental.pallas import tpu as pltpu
from jax.experimental.pallas import tpu_sc as plsc

NUM_EMB = 1000000
DIM = 32
TOTAL = 16384 * 50

_NC = 2
_NS = 16
_NW = _NC * _NS
_BPW = TOTAL // _NW
_CHUNK = 1280
_NCHUNK = _BPW // _CHUNK


def _emb_kernel(idx_hbm, table_hbm, out_hbm, idx_v, rows_a, rows_b,
                sem_idx, sem_g, sem_w):
    wid = lax.axis_index("s") * _NC + lax.axis_index("c")
    base = wid * _BPW
    pltpu.async_copy(idx_hbm.at[pl.ds(base, _BPW)], idx_v, sem_idx).wait()

    bufs = (rows_a, rows_b)
    writes = [None] * _NCHUNK
    for t in range(_NCHUNK):
        buf = bufs[t % 2]
        if t >= 2:
            writes[t - 2].wait()
        pltpu.async_copy(
            table_hbm.at[idx_v.at[pl.ds(t * _CHUNK, _CHUNK)]], buf, sem_g
        ).wait()
        writes[t] = pltpu.async_copy(
            buf, out_hbm.at[pl.ds(base + t * _CHUNK, _CHUNK)], sem_w)
    writes[_NCHUNK - 2].wait()
    writes[_NCHUNK - 1].wait()


@jax.jit
def _embedding_lookup(idx_flat, weight):
    mesh = plsc.VectorSubcoreMesh(core_axis_name="c", subcore_axis_name="s")
    f = functools.partial(
        pl.kernel,
        mesh=mesh,
        out_type=jax.ShapeDtypeStruct((TOTAL, DIM), jnp.float32),
        scratch_types=[
            pltpu.VMEM((_BPW,), jnp.int32),
            pltpu.VMEM((_CHUNK, DIM), jnp.float32),
            pltpu.VMEM((_CHUNK, DIM), jnp.float32),
            pltpu.SemaphoreType.DMA,
            pltpu.SemaphoreType.DMA,
            pltpu.SemaphoreType.DMA,
        ],
        compiler_params=pltpu.CompilerParams(use_tc_tiling_on_sc=False),
    )(_emb_kernel)
    return f(idx_flat, weight)


def kernel(x, weight):
    idx_flat = x.reshape(-1).astype(jnp.int32)
    out = _embedding_lookup(idx_flat, weight)
    return out.reshape(x.shape[0], x.shape[1], DIM)
